# baseline (speedup 1.0000x reference)
import functools

import jax
import jax.numpy as jnp
from jax import lax
from jax.experimental import pallas as pl
from jax.experimental.pallas import tpu as pltpu
from jax.experimental.pallas import tpu_sc as plsc

NU = 1024
DIM = 64
M = 100000
CHW = 128
NCH = 784
MP = NCH * CHW
BLK = 2048
NBLK = MP // BLK
CPB = BLK // CHW
TOPK = 10
NSEL = 16
NEG = -1024.0
NPAIR = 65536
NTILES = 32
PPT = NPAIR // NTILES
PROWS = PPT // CHW
GPT = (NU * NSEL) // NTILES
GROWS = GPT // CHW


NCHP = 896


def _mm_body(u_ref, it_ref, s_ref, c_ref, acc_ref):
    j = pl.program_id(0)
    s = lax.dot_general(u_ref[...], it_ref[...], (((1,), (1,)), ((), ())),
                        preferred_element_type=jnp.float32)
    col = j * BLK + lax.broadcasted_iota(jnp.int32, (NU, BLK), 1)
    s = jnp.where(col < M, s, NEG)
    s3 = s.reshape(NU, CPB, CHW)
    s_ref[...] = s3
    acc_ref[j % 8] = jnp.max(s3, axis=2)

    @pl.when(jnp.logical_or(j % 8 == 7, j == NBLK - 1))
    def _():
        cw = jnp.concatenate([acc_ref[k] for k in range(8)], axis=1)
        c_ref[...] = cw.reshape(NU, 1, 1, CHW)


def _matmul(users, items_p):
    return pl.pallas_call(
        _mm_body,
        grid=(NBLK,),
        in_specs=[
            pl.BlockSpec((NU, DIM), lambda j: (0, 0)),
            pl.BlockSpec((BLK, DIM), lambda j: (j, 0)),
        ],
        out_specs=[
            pl.BlockSpec((NU, CPB, CHW), lambda j: (0, j, 0)),
            pl.BlockSpec((NU, 1, 1, CHW), lambda j: (0, j // 8, 0, 0)),
        ],
        out_shape=(jax.ShapeDtypeStruct((NU, NCH, CHW), jnp.float32),
                   jax.ShapeDtypeStruct((NU, NCHP // CHW, 1, CHW),
                                        jnp.float32)),
        scratch_shapes=[pltpu.VMEM((8, NU, CPB), jnp.float32)],
        compiler_params=pltpu.CompilerParams(
            dimension_semantics=("arbitrary",)),
    )(users, items_p)


def _sc_scatter(scores_flat, idx3):
    mesh = plsc.VectorSubcoreMesh(core_axis_name="c", subcore_axis_name="s")

    def stateful(refs):
        s_ref, idx_ref = refs

        @pl.core_map(mesh)
        def _():
            def scoped(idx_v, val_v, sem):
                wid = lax.axis_index("s") * 2 + lax.axis_index("c")
                pltpu.sync_copy(idx_ref.at[wid], idx_v)
                for t in range(CHW // 16):
                    val_v[pl.ds(t * 16, 16)] = jnp.full((16,), NEG,
                                                        jnp.float32)
                cps = [pltpu.make_async_copy(val_v, s_ref.at[idx_v.at[j]],
                                             sem)
                       for j in range(PROWS)]
                for cp in cps:
                    cp.start()
                for cp in cps:
                    cp.wait()

            pl.run_scoped(scoped,
                          pltpu.VMEM((PROWS, CHW), jnp.int32),
                          pltpu.VMEM((CHW,), jnp.float32),
                          pltpu.SemaphoreType.DMA)

    scores_flat, _ = pl.run_state(stateful)((scores_flat, idx3))
    return scores_flat


def _sel_body(c_ref, q_ref):
    c = c_ref[...].reshape(NU, NCHP)[:, :NCH]
    cid = lax.broadcasted_iota(jnp.int32, (NU, NCH), 1)
    big_i = jnp.int32(2 ** 30)
    for k in range(NSEL):
        m = jnp.max(c, axis=1, keepdims=True)
        cand = jnp.where(c == m, cid, big_i)
        g = jnp.min(cand, axis=1, keepdims=True)
        q_ref[:, k:k + 1] = g
        c = jnp.where(cid == g, -jnp.inf, c)


def _select(c4):
    return pl.pallas_call(
        _sel_body,
        out_shape=jax.ShapeDtypeStruct((NU, NSEL), jnp.int32),
        compiler_params=pltpu.CompilerParams(
            vmem_limit_bytes=100 * 1024 * 1024),
    )(c4)


def _sc_gather(cidx3, sview):
    rows_per_tile = cidx3.shape[1] * CHW
    n_rows = NTILES * rows_per_tile
    r_idx = cidx3.shape[1]
    mesh = plsc.VectorSubcoreMesh(core_axis_name="c", subcore_axis_name="s")

    @functools.partial(
        pl.kernel,
        out_type=jax.ShapeDtypeStruct((n_rows, CHW), jnp.float32),
        mesh=mesh,
        scratch_types=[
            pltpu.VMEM((r_idx, CHW), jnp.int32),
            pltpu.VMEM((4 * CHW, CHW), jnp.float32),
            pltpu.SemaphoreType.DMA,
        ],
    )
    def k(cidx_hbm, sview_hbm, out_hbm, idx_v, buf, sem):
        wid = lax.axis_index("s") * 2 + lax.axis_index("c")
        pltpu.sync_copy(cidx_hbm.at[wid], idx_v)
        for g in range(r_idx // 4):
            cps = [pltpu.make_async_copy(
                       sview_hbm.at[idx_v.at[g * 4 + t]],
                       buf.at[pl.ds(t * CHW, CHW)], sem)
                   for t in range(4)]
            for cp in cps:
                cp.start()
            for cp in cps:
                cp.wait()
            pltpu.sync_copy(
                buf,
                out_hbm.at[pl.ds(wid * rows_per_tile + g * 4 * CHW,
                                 4 * CHW)])

    return k(cidx3, sview)


def _rowmax_body(x_ref, o_ref):
    o_ref[...] = jnp.max(x_ref[...], axis=1)


def _rowmax(x):
    n = x.shape[0]
    blk = 8192
    return pl.pallas_call(
        _rowmax_body,
        grid=(n // blk,),
        in_specs=[pl.BlockSpec((blk, CHW), lambda j: (j, 0))],
        out_specs=pl.BlockSpec((blk,), lambda j: (j,)),
        out_shape=jax.ShapeDtypeStruct((n,), jnp.float32),
    )(x)


def _sc_scatter_vals(dst_flat, idx3, val3):
    mesh = plsc.VectorSubcoreMesh(core_axis_name="c", subcore_axis_name="s")

    def stateful(refs):
        d_ref, idx_ref, val_ref = refs

        @pl.core_map(mesh)
        def _():
            def scoped(idx_v, val_v, sem):
                wid = lax.axis_index("s") * 2 + lax.axis_index("c")
                pltpu.sync_copy(idx_ref.at[wid], idx_v)
                pltpu.sync_copy(val_ref.at[wid], val_v)
                cps = [pltpu.make_async_copy(val_v.at[j],
                                             d_ref.at[idx_v.at[j]], sem)
                       for j in range(PROWS)]
                for cp in cps:
                    cp.start()
                for cp in cps:
                    cp.wait()

            pl.run_scoped(scoped,
                          pltpu.VMEM((PROWS, CHW), jnp.int32),
                          pltpu.VMEM((PROWS, CHW), jnp.float32),
                          pltpu.SemaphoreType.DMA)

    dst_flat, _, _ = pl.run_state(stateful)((dst_flat, idx3, val3))
    return dst_flat


def _top_body(v_ref, q_ref, tv_ref, ti_ref):
    v = v_ref[...].reshape(NU, NSEL * CHW)
    gi = (q_ref[...][:, :, None] * CHW
          + lax.broadcasted_iota(jnp.int32, (NU, NSEL, CHW), 2)
          ).reshape(NU, NSEL * CHW)
    big_i = jnp.int32(2 ** 30)
    for k in range(TOPK):
        m = jnp.max(v, axis=1, keepdims=True)
        cand = jnp.where(v == m, gi, big_i)
        g = jnp.min(cand, axis=1, keepdims=True)
        tv_ref[:, k:k + 1] = m
        ti_ref[:, k:k + 1] = g
        v = jnp.where(gi == g, -jnp.inf, v)


def _topk(vals, gidx):
    return pl.pallas_call(
        _top_body,
        out_shape=(jax.ShapeDtypeStruct((NU, TOPK), jnp.float32),
                   jax.ShapeDtypeStruct((NU, TOPK), jnp.int32)),
    )(vals, gidx)


def kernel(users_emb, items_emb, ignore_users, ignore_items):
    items_p = jnp.pad(items_emb, ((0, MP - M), (0, 0)))
    scores3, c4 = _matmul(users_emb, items_p)

    u = ignore_users.astype(jnp.int32)
    i = ignore_items.astype(jnp.int32)
    flat = u * MP + i
    scores_flat = _sc_scatter(scores3.reshape(NU * MP),
                              flat.reshape(NTILES, PROWS, CHW))
    sview = scores_flat.reshape(NU * NCH, CHW)

    cfr = lax.shift_right_logical(flat, 7)
    aff = _sc_gather(cfr.reshape(NTILES, PROWS, CHW), sview)
    rep = _rowmax(aff)
    rf = u * NCHP + lax.shift_right_logical(i, 7)
    c_flat = _sc_scatter_vals(c4.reshape(NU * NCHP),
                              rf.reshape(NTILES, PROWS, CHW),
                              rep.reshape(NTILES, PROWS, CHW))

    qid = _select(c_flat.reshape(NU, NCHP // CHW, 1, CHW))

    cidx = (jnp.arange(NU, dtype=jnp.int32)[:, None] * NCH + qid)
    gathered = _sc_gather(cidx.reshape(NTILES, GROWS, CHW), sview)

    pool3 = gathered.reshape(NU, NSEL, CHW)
    top_vals, top_items = _topk(pool3, qid)
    return top_vals, top_items
